# baseline (speedup 1.0000x reference)
import functools

import jax
import jax.numpy as jnp
from jax import lax
from jax.experimental import pallas as pl
from jax.experimental.pallas import tpu as pltpu
from jax.experimental.pallas import tpu_sc as plsc

_NC = 2
_NS = 16
_NW = _NC * _NS
_L = 16


def _edge_scatter_sc(n, e, mode):
    with_table = mode != "deg"
    k = 256
    epw = e // _NW
    epwp = -(-epw // k) * k
    ch = epwp // k
    npad = epwp - epw
    assert k % _L == 0 and n % _NS == 0 and npad % _L == 0 and e % _NW == 0
    rows_per_tile = n // _NS
    nbuf = 8
    dpre = 4
    assert ch > nbuf + dpre
    mesh = plsc.VectorSubcoreMesh(core_axis_name="c", subcore_axis_name="s")

    scratch = [
        pltpu.VMEM((epwp,), jnp.int32),
        pltpu.VMEM((epwp,), jnp.float32),
        pltpu.VMEM((ch, k), jnp.int32),
        pltpu.VMEM((nbuf, k, _L), jnp.float32),
        pltpu.VMEM((rows_per_tile, _L), jnp.float32),
        pltpu.VMEM_SHARED((n, _L), jnp.float32),
        [pltpu.SemaphoreType.DMA] * nbuf,
        [pltpu.SemaphoreType.DMA] * nbuf,
    ]
    if with_table:
        scratch.insert(0, pltpu.VMEM((epwp,), jnp.int32))
        scratch.append(pltpu.VMEM_SHARED((n, _L), jnp.float32))
        scratch.append(pltpu.VMEM((rows_per_tile, _L), jnp.float32))
    if mode == "msg_scale":
        scratch.append(pltpu.VMEM((rows_per_tile, _L), jnp.float32))
        scratch.append(pltpu.VMEM((rows_per_tile, _L), jnp.float32))

    def body(*refs):
        degp_hbm = table = h_v = d0_v = d1_v = None
        if mode == "msg_scale":
            (tab_hbm, degp_hbm, ei_hbm, ew_hbm, out_hbm,
             row_v, col_f, ew_v, col_v, msg_r, buf_v, acc, gsems, ssems,
             table, h_v, d0_v, d1_v) = refs
        elif mode == "msg_copy":
            (tab_hbm, ei_hbm, ew_hbm, out_hbm,
             row_v, col_f, ew_v, col_v, msg_r, buf_v, acc, gsems, ssems,
             table, h_v) = refs
        else:
            (ei_hbm, ew_hbm, out_hbm,
             col_f, ew_v, col_v, msg_r, buf_v, acc, gsems, ssems) = refs
        cid = lax.axis_index("c")
        sid = lax.axis_index("s")
        wid = sid * _NC + cid
        base = sid * rows_per_tile
        ebase = wid * epw

        if with_table:
            rsl = pl.ds(base, rows_per_tile)
            pltpu.sync_copy(tab_hbm.at[rsl], h_v)
            if mode == "msg_scale":
                pltpu.sync_copy(degp_hbm.at[0, rsl], d0_v)
                pltpu.sync_copy(degp_hbm.at[1, rsl], d1_v)

                @pl.loop(0, rows_per_tile)
                def _scalerow(i):
                    h_v[i] = h_v[i] / jnp.sqrt(d0_v[i] + d1_v[i] + 1.0)

            pltpu.sync_copy(h_v, table.at[rsl])

        @pl.loop(0, rows_per_tile)
        def _zero(i):
            buf_v[i] = jnp.zeros((_L,), jnp.float32)

        pltpu.sync_copy(buf_v, acc.at[pl.ds(base, rows_per_tile)])

        if with_table:
            pltpu.sync_copy(ei_hbm.at[0, pl.ds(ebase, epw)],
                            row_v.at[pl.ds(0, epw)])
        pltpu.sync_copy(ei_hbm.at[1, pl.ds(ebase, epw)],
                        col_f.at[pl.ds(0, epw)])
        pltpu.sync_copy(ew_hbm.at[pl.ds(ebase, epw)], ew_v.at[pl.ds(0, epw)])

        widv = jnp.broadcast_to(wid, (_L,)).astype(jnp.int32)
        for i in range(npad // _L):
            sl = pl.ds(epw + i * _L, _L)
            ew_v[sl] = jnp.zeros((_L,), jnp.float32)
            col_f[sl] = widv
            if with_table:
                row_v[sl] = widv

        @pl.loop(0, ch)
        def _stage(j):
            for g in range(k // _L):
                col_v[j, pl.ds(g * _L, _L)] = col_f[pl.ds(j * k + g * _L, _L)]

        plsc.subcore_barrier()

        def gather(j, u):
            if with_table:
                pltpu.async_copy(
                    table.at[row_v.at[pl.ds(j * k, k)]], msg_r.at[u], gsems[u])

        def gather_wait(j, u):
            if with_table:
                pltpu.make_async_copy(
                    table.at[row_v.at[pl.ds(j * k, k)]], msg_r.at[u],
                    gsems[u]).wait()

        def scale(j, u):
            buf = msg_r.at[u]
            for g in range(k // _L):
                ewv = ew_v[pl.ds(j * k + g * _L, _L)]
                for t in range(_L):
                    i = g * _L + t
                    if with_table:
                        buf[i] = buf[i] * ewv[t]
                    else:
                        buf[i] = jnp.broadcast_to(ewv[t], (_L,))

        def scatter(j, u):
            pltpu.async_copy(msg_r.at[u], acc.at[col_v.at[j]], ssems[u], add=True)

        def scatter_wait(j, u):
            pltpu.make_async_copy(msg_r.at[u], acc.at[col_v.at[j]], ssems[u]).wait()

        main_ch = ((ch - dpre) // nbuf) * nbuf

        for j in range(dpre):
            gather(j, j)

        @pl.loop(0, main_ch // nbuf)
        def _round(r):
            jr = r * nbuf
            for u in range(nbuf):
                j = jr + u
                uf = (u + dpre) % nbuf
                if u < dpre:
                    @pl.when(r > 0)
                    def _drain():
                        scatter_wait(j - dpre, uf)
                else:
                    scatter_wait(j - dpre, uf)
                gather(j + dpre, uf)
                gather_wait(j, u)
                scale(j, u)
                scatter(j, u)

        for j in range(main_ch, ch):
            u = j % nbuf
            uf = (j + dpre) % nbuf
            if j + dpre < ch:
                scatter_wait(j - dpre, uf)
                gather(j + dpre, uf)
            gather_wait(j, u)
            scale(j, u)
            scatter(j, u)

        for j in range(ch - nbuf, ch):
            scatter_wait(j, j % nbuf)

        plsc.subcore_barrier()
        pltpu.sync_copy(acc.at[pl.ds(base, rows_per_tile)], buf_v)
        pltpu.sync_copy(buf_v, out_hbm.at[cid, pl.ds(base, rows_per_tile)])

    return pl.kernel(
        body,
        out_type=jax.ShapeDtypeStruct((_NC, n, _L), jnp.float32),
        mesh=mesh,
        scratch_types=scratch,
        compiler_params=pltpu.CompilerParams(use_tc_tiling_on_sc=False),
    )


_PACK = 8


def _tc_mm_body(x_ref, wb_ref, o_ref):
    o_ref[...] = jnp.dot(x_ref[...], wb_ref[...],
                         preferred_element_type=jnp.float32)


def _tc_scale_body(degp_ref, h1_ref, h1p_ref, dinv_ref):
    deg = degp_ref[0] + degp_ref[1] + 1.0
    dinv = lax.rsqrt(deg)
    h1p_ref[...] = dinv * h1_ref[...]
    dinv_ref[...] = dinv


def _tc2_body(s1p_ref, dinv_ref, h1p_ref, b1_ref, w2b_ref, h2p_ref):
    dinv = dinv_ref[...]
    pre = dinv * (s1p_ref[0] + s1p_ref[1] + h1p_ref[...]) + b1_ref[...]
    out1 = jnp.maximum(pre, 0.0)
    h2 = jnp.dot(out1, w2b_ref[...], preferred_element_type=jnp.float32)
    h2p_ref[...] = dinv * h2


def _tc3_body(s2p_ref, dinv_ref, h2p_ref, b2_ref, gs_ref, gb_ref, y_ref):
    pre = dinv_ref[...] * (s2p_ref[0] + s2p_ref[1] + h2p_ref[...]) + b2_ref[...]
    e = jnp.exp(pre)
    s = jnp.dot(e, gs_ref[...], preferred_element_type=jnp.float32,
                precision=lax.Precision.HIGHEST)
    lsb = jnp.dot(jnp.log(s), gb_ref[...], preferred_element_type=jnp.float32,
                  precision=lax.Precision.HIGHEST)
    y_ref[...] = pre - lsb


def _tc(body, out_shapes):
    return pl.pallas_call(body, out_shape=out_shapes)


@functools.partial(jax.jit, static_argnames=())
def kernel(x, edge_index, edge_weight, W1, b1, W2, b2):
    n, f_in = x.shape
    e = edge_weight.shape[0]
    np_ = n // _PACK
    w = _PACK * _L
    fp = jax.ShapeDtypeStruct((np_, w), jnp.float32)

    eye8 = jnp.eye(_PACK, dtype=jnp.float32)
    w1b = jnp.kron(eye8, W1)
    w2b = jnp.kron(eye8, W2)
    b1t = jnp.tile(b1, _PACK).reshape(1, w)
    b2t = jnp.tile(b2, _PACK).reshape(1, w)
    gids = jnp.arange(w, dtype=jnp.int32) // _L
    gs = (gids[:, None] == jnp.arange(_PACK, dtype=jnp.int32)[None, :]
          ).astype(jnp.float32)
    gb = gs.T
    x8 = x.reshape(np_, _PACK * f_in)

    deg_sc = _edge_scatter_sc(n, e, mode="deg")
    msg_sc = _edge_scatter_sc(n, e, mode="msg_copy")

    degp = deg_sc(edge_index, edge_weight)
    h1_8 = _tc(_tc_mm_body, fp)(x8, w1b)
    h1p8, dinv8 = _tc(_tc_scale_body, (fp, fp))(
        degp.reshape(_NC, np_, w), h1_8)
    s1p = msg_sc(h1p8.reshape(n, _L), edge_index, edge_weight)
    h2p8 = _tc(_tc2_body, fp)(
        s1p.reshape(_NC, np_, w), dinv8, h1p8, b1t, w2b)
    s2p = msg_sc(h2p8.reshape(n, _L), edge_index, edge_weight)
    y8 = _tc(_tc3_body, fp)(
        s2p.reshape(_NC, np_, w), dinv8, h2p8, b2t, gs, gb)
    return y8.reshape(n, _L)


# speedup vs baseline: 1.3424x; 1.3424x over previous
import functools

import jax
import jax.numpy as jnp
from jax import lax
from jax.experimental import pallas as pl
from jax.experimental.pallas import tpu as pltpu
from jax.experimental.pallas import tpu_sc as plsc

_NC = 2
_NS = 16
_NW = _NC * _NS
_L = 16


def _edge_scatter_sc(n, e, mode):
    with_table = mode != "deg"
    k = 80
    epw = e // _NW
    epwp = -(-epw // k) * k
    ch = epwp // k
    npad = epwp - epw
    assert k % _L == 0 and n % _NS == 0 and npad % _L == 0 and e % _NW == 0
    rows_per_tile = n // _NS
    nbuf = 8
    dpre = 4
    assert ch > nbuf + dpre
    mesh = plsc.VectorSubcoreMesh(core_axis_name="c", subcore_axis_name="s")

    scratch = [
        pltpu.VMEM((epwp,), jnp.int32),
        pltpu.VMEM((epwp,), jnp.float32),
        pltpu.VMEM((ch, k), jnp.int32),
        pltpu.VMEM((nbuf, k, _L), jnp.float32),
        pltpu.VMEM((rows_per_tile, _L), jnp.float32),
        pltpu.VMEM_SHARED((n, _L), jnp.float32),
        [pltpu.SemaphoreType.DMA] * nbuf,
        [pltpu.SemaphoreType.DMA] * nbuf,
    ]
    if with_table:
        scratch.insert(0, pltpu.VMEM((epwp,), jnp.int32))
        scratch.append(pltpu.VMEM_SHARED((n, _L), jnp.float32))
        scratch.append(pltpu.VMEM((rows_per_tile, _L), jnp.float32))
    if mode == "msg_scale":
        scratch.append(pltpu.VMEM((rows_per_tile, _L), jnp.float32))
        scratch.append(pltpu.VMEM((rows_per_tile, _L), jnp.float32))

    def body(*refs):
        degp_hbm = table = h_v = d0_v = d1_v = None
        if mode == "msg_scale":
            (tab_hbm, degp_hbm, ei_hbm, ew_hbm, out_hbm,
             row_v, col_f, ew_v, col_v, msg_r, buf_v, acc, gsems, ssems,
             table, h_v, d0_v, d1_v) = refs
        elif mode == "msg_copy":
            (tab_hbm, ei_hbm, ew_hbm, out_hbm,
             row_v, col_f, ew_v, col_v, msg_r, buf_v, acc, gsems, ssems,
             table, h_v) = refs
        else:
            (ei_hbm, ew_hbm, out_hbm,
             col_f, ew_v, col_v, msg_r, buf_v, acc, gsems, ssems) = refs
        cid = lax.axis_index("c")
        sid = lax.axis_index("s")
        wid = sid * _NC + cid
        base = sid * rows_per_tile
        ebase = wid * epw

        if with_table:
            rsl = pl.ds(base, rows_per_tile)
            pltpu.sync_copy(tab_hbm.at[rsl], h_v)
            if mode == "msg_scale":
                pltpu.sync_copy(degp_hbm.at[0, rsl], d0_v)
                pltpu.sync_copy(degp_hbm.at[1, rsl], d1_v)

                @pl.loop(0, rows_per_tile)
                def _scalerow(i):
                    h_v[i] = h_v[i] / jnp.sqrt(d0_v[i] + d1_v[i] + 1.0)

            pltpu.sync_copy(h_v, table.at[rsl])

        @pl.loop(0, rows_per_tile)
        def _zero(i):
            buf_v[i] = jnp.zeros((_L,), jnp.float32)

        pltpu.sync_copy(buf_v, acc.at[pl.ds(base, rows_per_tile)])

        if with_table:
            pltpu.sync_copy(ei_hbm.at[0, pl.ds(ebase, epw)],
                            row_v.at[pl.ds(0, epw)])
        pltpu.sync_copy(ei_hbm.at[1, pl.ds(ebase, epw)],
                        col_f.at[pl.ds(0, epw)])
        pltpu.sync_copy(ew_hbm.at[pl.ds(ebase, epw)], ew_v.at[pl.ds(0, epw)])

        widv = jnp.broadcast_to(wid, (_L,)).astype(jnp.int32)
        for i in range(npad // _L):
            sl = pl.ds(epw + i * _L, _L)
            ew_v[sl] = jnp.zeros((_L,), jnp.float32)
            col_f[sl] = widv
            if with_table:
                row_v[sl] = widv

        @pl.loop(0, ch)
        def _stage(j):
            for g in range(k // _L):
                col_v[j, pl.ds(g * _L, _L)] = col_f[pl.ds(j * k + g * _L, _L)]

        plsc.subcore_barrier()

        def gather(j, u):
            if with_table:
                pltpu.async_copy(
                    table.at[row_v.at[pl.ds(j * k, k)]], msg_r.at[u], gsems[u])

        def gather_wait(j, u):
            if with_table:
                pltpu.make_async_copy(
                    table.at[row_v.at[pl.ds(j * k, k)]], msg_r.at[u],
                    gsems[u]).wait()

        def scale(j, u):
            buf = msg_r.at[u]
            for g in range(k // _L):
                ewv = ew_v[pl.ds(j * k + g * _L, _L)]
                for t in range(_L):
                    i = g * _L + t
                    if with_table:
                        buf[i] = buf[i] * ewv[t]
                    else:
                        buf[i] = jnp.broadcast_to(ewv[t], (_L,))

        def scatter(j, u):
            pltpu.async_copy(msg_r.at[u], acc.at[col_v.at[j]], ssems[u], add=True)

        def scatter_wait(j, u):
            pltpu.make_async_copy(msg_r.at[u], acc.at[col_v.at[j]], ssems[u]).wait()

        main_ch = ((ch - dpre) // nbuf) * nbuf

        for j in range(dpre):
            gather(j, j)

        @pl.loop(0, main_ch // nbuf)
        def _round(r):
            jr = r * nbuf
            for u in range(nbuf):
                j = jr + u
                uf = (u + dpre) % nbuf
                if u < dpre:
                    @pl.when(r > 0)
                    def _drain():
                        scatter_wait(j - dpre, uf)
                else:
                    scatter_wait(j - dpre, uf)
                gather(j + dpre, uf)
                gather_wait(j, u)
                scale(j, u)
                scatter(j, u)

        for j in range(main_ch, ch):
            u = j % nbuf
            uf = (j + dpre) % nbuf
            if j + dpre < ch:
                scatter_wait(j - dpre, uf)
                gather(j + dpre, uf)
            gather_wait(j, u)
            scale(j, u)
            scatter(j, u)

        for j in range(ch - nbuf, ch):
            scatter_wait(j, j % nbuf)

        plsc.subcore_barrier()
        pltpu.sync_copy(acc.at[pl.ds(base, rows_per_tile)], buf_v)
        pltpu.sync_copy(buf_v, out_hbm.at[cid, pl.ds(base, rows_per_tile)])

    return pl.kernel(
        body,
        out_type=jax.ShapeDtypeStruct((_NC, n, _L), jnp.float32),
        mesh=mesh,
        scratch_types=scratch,
        compiler_params=pltpu.CompilerParams(use_tc_tiling_on_sc=False),
    )


_PACK = 8


def _tc_mm_body(x_ref, wb_ref, o_ref):
    o_ref[...] = jnp.dot(x_ref[...], wb_ref[...],
                         preferred_element_type=jnp.float32)


def _tc_scale_body(degp_ref, h1_ref, h1p_ref, dinv_ref):
    deg = degp_ref[0] + degp_ref[1] + 1.0
    dinv = lax.rsqrt(deg)
    h1p_ref[...] = dinv * h1_ref[...]
    dinv_ref[...] = dinv


def _tc2_body(s1p_ref, dinv_ref, h1p_ref, b1_ref, w2b_ref, h2p_ref):
    dinv = dinv_ref[...]
    pre = dinv * (s1p_ref[0] + s1p_ref[1] + h1p_ref[...]) + b1_ref[...]
    out1 = jnp.maximum(pre, 0.0)
    h2 = jnp.dot(out1, w2b_ref[...], preferred_element_type=jnp.float32)
    h2p_ref[...] = dinv * h2


def _tc3_body(s2p_ref, dinv_ref, h2p_ref, b2_ref, gs_ref, gb_ref, y_ref):
    pre = dinv_ref[...] * (s2p_ref[0] + s2p_ref[1] + h2p_ref[...]) + b2_ref[...]
    e = jnp.exp(pre)
    s = jnp.dot(e, gs_ref[...], preferred_element_type=jnp.float32,
                precision=lax.Precision.HIGHEST)
    lsb = jnp.dot(jnp.log(s), gb_ref[...], preferred_element_type=jnp.float32,
                  precision=lax.Precision.HIGHEST)
    y_ref[...] = pre - lsb


def _tc(body, out_shapes):
    return pl.pallas_call(body, out_shape=out_shapes)


@functools.partial(jax.jit, static_argnames=())
def kernel(x, edge_index, edge_weight, W1, b1, W2, b2):
    n, f_in = x.shape
    e = edge_weight.shape[0]
    np_ = n // _PACK
    w = _PACK * _L
    fp = jax.ShapeDtypeStruct((np_, w), jnp.float32)

    eye8 = jnp.eye(_PACK, dtype=jnp.float32)
    w1b = jnp.kron(eye8, W1)
    w2b = jnp.kron(eye8, W2)
    b1t = jnp.tile(b1, _PACK).reshape(1, w)
    b2t = jnp.tile(b2, _PACK).reshape(1, w)
    gids = jnp.arange(w, dtype=jnp.int32) // _L
    gs = (gids[:, None] == jnp.arange(_PACK, dtype=jnp.int32)[None, :]
          ).astype(jnp.float32)
    gb = gs.T
    x8 = x.reshape(np_, _PACK * f_in)

    deg_sc = _edge_scatter_sc(n, e, mode="deg")
    msg_sc = _edge_scatter_sc(n, e, mode="msg_copy")

    degp = deg_sc(edge_index, edge_weight)
    h1_8 = _tc(_tc_mm_body, fp)(x8, w1b)
    h1p8, dinv8 = _tc(_tc_scale_body, (fp, fp))(
        degp.reshape(_NC, np_, w), h1_8)
    s1p = msg_sc(h1p8.reshape(n, _L), edge_index, edge_weight)
    h2p8 = _tc(_tc2_body, fp)(
        s1p.reshape(_NC, np_, w), dinv8, h1p8, b1t, w2b)
    s2p = msg_sc(h2p8.reshape(n, _L), edge_index, edge_weight)
    y8 = _tc(_tc3_body, fp)(
        s2p.reshape(_NC, np_, w), dinv8, h2p8, b2t, gs, gb)
    return y8.reshape(n, _L)
